# baseline (speedup 1.0000x reference)
import functools

import jax
import jax.numpy as jnp
from jax import lax
from jax.experimental import pallas as pl
from jax.experimental.pallas import tpu as pltpu
from jax.experimental.pallas import tpu_sc as plsc

_NC = 2
_NS = 16
_NW = _NC * _NS
_LANES = 16

_BR = 512


def _sc_gather(inputs, gflat, ss2d, n, kk, d_in):
    nk = n * kk
    rows_per_w = nk // _NW
    cw = 32
    ch = rows_per_w // cw
    nbuf = 4
    mesh = plsc.VectorSubcoreMesh(core_axis_name="c", subcore_axis_name="s")

    @functools.partial(
        pl.kernel,
        out_type=[
            jax.ShapeDtypeStruct((nk, d_in), jnp.float32),
            jax.ShapeDtypeStruct((nk // cw, cw), jnp.float32),
            jax.ShapeDtypeStruct((nk,), jnp.int32),
        ],
        mesh=mesh,
        scratch_types=[
            pltpu.VMEM((ch, cw), jnp.int32),
            pltpu.VMEM((ch, cw), jnp.int32),
            pltpu.VMEM((ch, cw), jnp.int32),
            [pltpu.VMEM((cw, d_in), jnp.float32) for _ in range(nbuf)],
            pltpu.VMEM((ch, cw), jnp.float32),
            [pltpu.SemaphoreType.DMA for _ in range(nbuf)],
            [pltpu.SemaphoreType.DMA for _ in range(nbuf)],
            pltpu.SemaphoreType.DMA,
            pltpu.SemaphoreType.DMA,
        ],
    )
    def k(ss_hbm, in_hbm, gates_hbm, xg_hbm, gg_hbm, inv_hbm,
          idx2d, tok2d, val2d, rows, gg2d, semg, semw, semgg, seminv):
        wid = lax.axis_index("s") * _NC + lax.axis_index("c")
        base = wid * rows_per_w
        iota16 = lax.iota(jnp.int32, 16)
        shift = jnp.full((16,), kk.bit_length() - 1, jnp.int32)
        pltpu.sync_copy(ss_hbm.at[pl.ds(wid * ch, ch)], idx2d)
        for j in range(ch):
            for c in range(cw // 16):
                sl = pl.ds(c * 16, 16)
                v = idx2d[j, sl]
                tok2d[j, sl] = lax.shift_right_logical(v, shift)
                val2d[j, sl] = base + j * cw + c * 16 + iota16
        gg_cps = [pltpu.async_copy(gates_hbm.at[idx2d.at[it]], gg2d.at[it], semgg)
                  for it in range(ch)]
        inv_cps = [pltpu.async_copy(val2d.at[it], inv_hbm.at[idx2d.at[it]], seminv)
                   for it in range(ch)]
        g_cp = [None] * nbuf
        w_cp = [None] * nbuf

        def gather(it):
            return pltpu.async_copy(in_hbm.at[tok2d.at[it]], rows[it % nbuf],
                                    semg[it % nbuf])

        for it in range(nbuf - 1):
            g_cp[it % nbuf] = gather(it)
        for it in range(ch):
            b = it % nbuf
            ia = it + nbuf - 1
            if ia < ch:
                ba = ia % nbuf
                if w_cp[ba] is not None:
                    w_cp[ba].wait()
                    w_cp[ba] = None
                g_cp[ba] = gather(ia)
            g_cp[b].wait()
            w_cp[b] = pltpu.async_copy(rows[b], xg_hbm.at[pl.ds(base + it * cw, cw)],
                                       semw[b])
        for b in range(nbuf):
            if w_cp[b] is not None:
                w_cp[b].wait()
        for cp in gg_cps:
            cp.wait()
        for cp in inv_cps:
            cp.wait()
        pltpu.sync_copy(gg2d, gg_hbm.at[pl.ds(wid * ch, ch)])

    return k(ss2d, inputs, gflat)


def _mm_body(bid_ref, eid_ref, s0_ref, s1_ref, x_ref, w_ref, g_ref, o_ref, acc):
    g = pl.num_programs(0)
    p = pl.program_id(0)
    blk = bid_ref[p]
    st = s0_ref[p]
    en = s1_ref[p]
    rows = blk * _BR + lax.broadcasted_iota(jnp.int32, (_BR,), 0)
    m = (rows >= st) & (rows < en)
    gm = jnp.where(m, g_ref[0, 0, :], 0.0)
    y = lax.dot_general(x_ref[...].astype(jnp.bfloat16),
                        w_ref[0].astype(jnp.bfloat16),
                        (((1,), (1,)), ((), ())),
                        preferred_element_type=jnp.float32)
    y = y * gm[:, None]

    prev = bid_ref[jnp.maximum(p - 1, 0)]
    first = jnp.logical_or(p == 0, blk != prev)
    nxt = bid_ref[jnp.minimum(p + 1, g - 1)]
    last = jnp.logical_or(p == g - 1, blk != nxt)

    @pl.when(first)
    def _():
        acc[...] = y

    @pl.when(jnp.logical_not(first))
    def _():
        acc[...] += y

    @pl.when(last)
    def _():
        o_ref[...] = acc[...]


def _grouped_mm(xg, weight, gg, expert_offsets, nk, d_in, d_out, e):
    nb = nk // _BR
    g = nb + e

    off = expert_offsets.astype(jnp.int32)
    start = jnp.concatenate([jnp.zeros((1,), jnp.int32), off[:-1]])
    end = off
    start_m = jnp.concatenate([off[-1:], off[:-1]])
    tiles = jnp.where(end > start, (end - 1) // _BR - start // _BR + 1, 0)
    cum = jnp.cumsum(tiles)
    pair_start = cum - tiles
    total = cum[-1]
    p = jnp.arange(g, dtype=jnp.int32)
    e_p = jnp.searchsorted(cum, p, side="right").astype(jnp.int32)
    e_c = jnp.minimum(e_p, e - 1)
    bid = start[e_c] // _BR + (p - pair_start[e_c])
    bid = jnp.clip(bid, 0, nb - 1).astype(jnp.int32)
    valid = p < total
    s0 = jnp.where(valid, start_m[e_c], 0).astype(jnp.int32)
    s1 = jnp.where(valid, end[e_c], 0).astype(jnp.int32)

    gg3 = gg.reshape(nb, 1, _BR)

    grid_spec = pltpu.PrefetchScalarGridSpec(
        num_scalar_prefetch=4,
        grid=(g,),
        in_specs=[
            pl.BlockSpec((_BR, d_in), lambda p, bid, eid, s0, s1: (bid[p], 0)),
            pl.BlockSpec((1, d_out, d_in),
                         lambda p, bid, eid, s0, s1: (eid[p], 0, 0)),
            pl.BlockSpec((1, 1, _BR), lambda p, bid, eid, s0, s1: (bid[p], 0, 0)),
        ],
        out_specs=pl.BlockSpec((_BR, d_out), lambda p, bid, eid, s0, s1: (bid[p], 0)),
        scratch_shapes=[pltpu.VMEM((_BR, d_out), jnp.float32)],
    )
    return pl.pallas_call(
        _mm_body,
        grid_spec=grid_spec,
        out_shape=jax.ShapeDtypeStruct((nk, d_out), jnp.float32),
    )(bid, e_c, s0, s1, xg, weight, gg3)


def _sc_combine(yg, invA2d, invB2d, n, kk, d_out):
    toks_per_w = n // _NW
    ct = 32
    iters = toks_per_w // ct
    mesh = plsc.VectorSubcoreMesh(core_axis_name="c", subcore_axis_name="s")

    @functools.partial(
        pl.kernel,
        out_type=jax.ShapeDtypeStruct((n, d_out), jnp.float32),
        mesh=mesh,
        scratch_types=[
            pltpu.VMEM((iters, 32), jnp.int32),
            pltpu.VMEM((iters, 32), jnp.int32),
            pltpu.VMEM((ct, d_out), jnp.float32),
            pltpu.VMEM((ct, d_out), jnp.float32),
            pltpu.VMEM((ct, d_out), jnp.float32),
            pltpu.VMEM((ct, d_out), jnp.float32),
            pltpu.SemaphoreType.DMA,
            pltpu.SemaphoreType.DMA,
            pltpu.SemaphoreType.DMA,
            pltpu.SemaphoreType.DMA,
            pltpu.SemaphoreType.DMA,
            pltpu.SemaphoreType.DMA,
        ],
    )
    def k(yg_hbm, invA_hbm, invB_hbm, out_hbm, invA_v, invB_v,
          rows_a, rows_b, out_a, out_b,
          semga_a, semga_b, semgb_a, semgb_b, semw_a, semw_b):
        wid = lax.axis_index("s") * _NC + lax.axis_index("c")
        pltpu.sync_copy(invA_hbm.at[pl.ds(wid * iters, iters)], invA_v)
        pltpu.sync_copy(invB_hbm.at[pl.ds(wid * iters, iters)], invB_v)
        rows = [rows_a, rows_b]
        outs = [out_a, out_b]
        semga = [semga_a, semga_b]
        semgb = [semgb_a, semgb_b]
        semw = [semw_a, semw_b]
        ga_cp = [None, None]
        gb_cp = [None, None]
        w_cp = [None, None]
        ga_cp[0] = pltpu.async_copy(yg_hbm.at[invA_v.at[0]], outs[0], semga[0])
        gb_cp[0] = pltpu.async_copy(yg_hbm.at[invB_v.at[0]], rows[0], semgb[0])
        for it in range(iters):
            b = it % 2
            nb = (it + 1) % 2
            if it + 1 < iters:
                if w_cp[nb] is not None:
                    w_cp[nb].wait()
                ga_cp[nb] = pltpu.async_copy(yg_hbm.at[invA_v.at[it + 1]],
                                             outs[nb], semga[nb])
                gb_cp[nb] = pltpu.async_copy(yg_hbm.at[invB_v.at[it + 1]],
                                             rows[nb], semgb[nb])
            ga_cp[b].wait()
            gb_cp[b].wait()
            rows_v = rows[b]
            out_v = outs[b]

            def tok_body(t, carry):
                for c in range(d_out // 16):
                    sl = pl.ds(c * 16, 16)
                    plsc.addupdate(out_v.at[t, sl], rows_v[t, sl])
                return carry

            lax.fori_loop(0, ct, tok_body, 0)
            w_cp[b] = pltpu.async_copy(
                out_v, out_hbm.at[pl.ds(wid * toks_per_w + it * ct, ct)], semw[b])
        for b in range(2):
            if w_cp[b] is not None:
                w_cp[b].wait()

    return k(yg, invA2d, invB2d)


def kernel(inputs, weight, gates, k, sorted_expert_idxs, sorted_scattered_idxs,
           padded_block_idxs, expert_offsets):
    n, d_in = inputs.shape
    e, d_out, _ = weight.shape
    kk = gates.shape[1]
    nk = n * kk

    ss2d = sorted_scattered_idxs.astype(jnp.int32).reshape(nk // 32, 32)
    xg, gg2d, inv = _sc_gather(inputs, gates.reshape(nk), ss2d, n, kk, d_in)
    yg = _grouped_mm(xg, weight, gg2d.reshape(nk), expert_offsets,
                     nk, d_in, d_out, e)
    inv_nk = inv.reshape(n, kk)
    invA2d = inv_nk[:, 0].reshape(n // 32, 32)
    invB2d = inv_nk[:, 1].reshape(n // 32, 32)
    return _sc_combine(yg, invA2d, invB2d, n, kk, d_out)


# speedup vs baseline: 1.0113x; 1.0113x over previous
import functools

import jax
import jax.numpy as jnp
from jax import lax
from jax.experimental import pallas as pl
from jax.experimental.pallas import tpu as pltpu
from jax.experimental.pallas import tpu_sc as plsc

_NC = 2
_NS = 16
_NW = _NC * _NS
_LANES = 16

_BR = 512


def _sc_gather(inputs, gflat, ss2d, n, kk, d_in):
    nk = n * kk
    rows_per_w = nk // _NW
    cw = 32
    ch = rows_per_w // cw
    nbuf = 4
    mesh = plsc.VectorSubcoreMesh(core_axis_name="c", subcore_axis_name="s")

    @functools.partial(
        pl.kernel,
        out_type=[
            jax.ShapeDtypeStruct((nk, d_in), jnp.float32),
            jax.ShapeDtypeStruct((nk // cw, cw), jnp.float32),
            jax.ShapeDtypeStruct((nk,), jnp.int32),
        ],
        mesh=mesh,
        scratch_types=[
            pltpu.VMEM((ch, cw), jnp.int32),
            pltpu.VMEM((ch, cw), jnp.int32),
            pltpu.VMEM((ch, cw), jnp.int32),
            [pltpu.VMEM((cw, d_in), jnp.float32) for _ in range(nbuf)],
            pltpu.VMEM((ch, cw), jnp.float32),
            [pltpu.SemaphoreType.DMA for _ in range(nbuf)],
            [pltpu.SemaphoreType.DMA for _ in range(nbuf)],
            pltpu.SemaphoreType.DMA,
            pltpu.SemaphoreType.DMA,
        ],
    )
    def k(ss_hbm, in_hbm, gates_hbm, xg_hbm, gg_hbm, inv_hbm,
          idx2d, tok2d, val2d, rows, gg2d, semg, semw, semgg, seminv):
        wid = lax.axis_index("s") * _NC + lax.axis_index("c")
        base = wid * rows_per_w
        iota16 = lax.iota(jnp.int32, 16)
        shift = jnp.full((16,), kk.bit_length() - 1, jnp.int32)
        pltpu.sync_copy(ss_hbm.at[pl.ds(wid * ch, ch)], idx2d)
        for j in range(ch):
            for c in range(cw // 16):
                sl = pl.ds(c * 16, 16)
                v = idx2d[j, sl]
                tok2d[j, sl] = lax.shift_right_logical(v, shift)
                val2d[j, sl] = base + j * cw + c * 16 + iota16
        gg_cps = [pltpu.async_copy(gates_hbm.at[idx2d.at[it]], gg2d.at[it], semgg)
                  for it in range(ch)]
        inv_cps = [pltpu.async_copy(val2d.at[it], inv_hbm.at[idx2d.at[it]], seminv)
                   for it in range(ch)]
        g_cp = [None] * nbuf
        w_cp = [None] * nbuf

        def gather(it):
            return pltpu.async_copy(in_hbm.at[tok2d.at[it]], rows[it % nbuf],
                                    semg[it % nbuf])

        for it in range(nbuf - 1):
            g_cp[it % nbuf] = gather(it)
        for it in range(ch):
            b = it % nbuf
            ia = it + nbuf - 1
            if ia < ch:
                ba = ia % nbuf
                if w_cp[ba] is not None:
                    w_cp[ba].wait()
                    w_cp[ba] = None
                g_cp[ba] = gather(ia)
            g_cp[b].wait()
            w_cp[b] = pltpu.async_copy(rows[b], xg_hbm.at[pl.ds(base + it * cw, cw)],
                                       semw[b])
        for b in range(nbuf):
            if w_cp[b] is not None:
                w_cp[b].wait()
        for cp in gg_cps:
            cp.wait()
        for cp in inv_cps:
            cp.wait()
        pltpu.sync_copy(gg2d, gg_hbm.at[pl.ds(wid * ch, ch)])

    return k(ss2d, inputs, gflat)


def _mm_body(bid_ref, eid_ref, s0_ref, s1_ref, x_ref, w_ref, g_ref, o_ref):
    p = pl.program_id(0)
    blk = bid_ref[p]
    st = s0_ref[p]
    en = s1_ref[p]
    rows = blk * _BR + lax.broadcasted_iota(jnp.int32, (_BR,), 0)
    m = (rows >= st) & (rows < en)
    gm = jnp.where(m, g_ref[0, 0, :], 0.0)
    y = lax.dot_general(x_ref[...].astype(jnp.bfloat16),
                        w_ref[0].astype(jnp.bfloat16),
                        (((1,), (1,)), ((), ())),
                        preferred_element_type=jnp.float32)
    y = y * gm[:, None]

    prev = bid_ref[jnp.maximum(p - 1, 0)]
    first = jnp.logical_or(p == 0, blk != prev)

    @pl.when(first)
    def _():
        o_ref[...] = jnp.zeros_like(o_ref)

    o_ref[...] += y


def _grouped_mm(xg, weight, gg, expert_offsets, nk, d_in, d_out, e):
    nb = nk // _BR
    g = nb + e

    off = expert_offsets.astype(jnp.int32)
    start = jnp.concatenate([jnp.zeros((1,), jnp.int32), off[:-1]])
    end = off
    start_m = jnp.concatenate([off[-1:], off[:-1]])
    tiles = jnp.where(end > start, (end - 1) // _BR - start // _BR + 1, 0)
    cum = jnp.cumsum(tiles)
    pair_start = cum - tiles
    total = cum[-1]
    p = jnp.arange(g, dtype=jnp.int32)
    e_p = jnp.searchsorted(cum, p, side="right").astype(jnp.int32)
    e_c = jnp.minimum(e_p, e - 1)
    bid = start[e_c] // _BR + (p - pair_start[e_c])
    bid = jnp.clip(bid, 0, nb - 1).astype(jnp.int32)
    valid = p < total
    s0 = jnp.where(valid, start_m[e_c], 0).astype(jnp.int32)
    s1 = jnp.where(valid, end[e_c], 0).astype(jnp.int32)

    gg3 = gg.reshape(nb, 1, _BR)

    grid_spec = pltpu.PrefetchScalarGridSpec(
        num_scalar_prefetch=4,
        grid=(g,),
        in_specs=[
            pl.BlockSpec((_BR, d_in), lambda p, bid, eid, s0, s1: (bid[p], 0)),
            pl.BlockSpec((1, d_out, d_in),
                         lambda p, bid, eid, s0, s1: (eid[p], 0, 0)),
            pl.BlockSpec((1, 1, _BR), lambda p, bid, eid, s0, s1: (bid[p], 0, 0)),
        ],
        out_specs=pl.BlockSpec((_BR, d_out), lambda p, bid, eid, s0, s1: (bid[p], 0)),
    )
    return pl.pallas_call(
        _mm_body,
        grid_spec=grid_spec,
        out_shape=jax.ShapeDtypeStruct((nk, d_out), jnp.float32),
    )(bid, e_c, s0, s1, xg, weight, gg3)


def _sc_combine(yg, invA2d, invB2d, n, kk, d_out):
    toks_per_w = n // _NW
    ct = 32
    iters = toks_per_w // ct
    mesh = plsc.VectorSubcoreMesh(core_axis_name="c", subcore_axis_name="s")

    @functools.partial(
        pl.kernel,
        out_type=jax.ShapeDtypeStruct((n, d_out), jnp.float32),
        mesh=mesh,
        scratch_types=[
            pltpu.VMEM((iters, 32), jnp.int32),
            pltpu.VMEM((iters, 32), jnp.int32),
            pltpu.VMEM((ct, d_out), jnp.float32),
            pltpu.VMEM((ct, d_out), jnp.float32),
            pltpu.VMEM((ct, d_out), jnp.float32),
            pltpu.VMEM((ct, d_out), jnp.float32),
            pltpu.SemaphoreType.DMA,
            pltpu.SemaphoreType.DMA,
            pltpu.SemaphoreType.DMA,
            pltpu.SemaphoreType.DMA,
            pltpu.SemaphoreType.DMA,
            pltpu.SemaphoreType.DMA,
        ],
    )
    def k(yg_hbm, invA_hbm, invB_hbm, out_hbm, invA_v, invB_v,
          rows_a, rows_b, out_a, out_b,
          semga_a, semga_b, semgb_a, semgb_b, semw_a, semw_b):
        wid = lax.axis_index("s") * _NC + lax.axis_index("c")
        pltpu.sync_copy(invA_hbm.at[pl.ds(wid * iters, iters)], invA_v)
        pltpu.sync_copy(invB_hbm.at[pl.ds(wid * iters, iters)], invB_v)
        rows = [rows_a, rows_b]
        outs = [out_a, out_b]
        semga = [semga_a, semga_b]
        semgb = [semgb_a, semgb_b]
        semw = [semw_a, semw_b]
        ga_cp = [None, None]
        gb_cp = [None, None]
        w_cp = [None, None]
        ga_cp[0] = pltpu.async_copy(yg_hbm.at[invA_v.at[0]], outs[0], semga[0])
        gb_cp[0] = pltpu.async_copy(yg_hbm.at[invB_v.at[0]], rows[0], semgb[0])
        for it in range(iters):
            b = it % 2
            nb = (it + 1) % 2
            if it + 1 < iters:
                if w_cp[nb] is not None:
                    w_cp[nb].wait()
                ga_cp[nb] = pltpu.async_copy(yg_hbm.at[invA_v.at[it + 1]],
                                             outs[nb], semga[nb])
                gb_cp[nb] = pltpu.async_copy(yg_hbm.at[invB_v.at[it + 1]],
                                             rows[nb], semgb[nb])
            ga_cp[b].wait()
            gb_cp[b].wait()
            rows_v = rows[b]
            out_v = outs[b]

            def tok_body(t, carry):
                for c in range(d_out // 16):
                    sl = pl.ds(c * 16, 16)
                    plsc.addupdate(out_v.at[t, sl], rows_v[t, sl])
                return carry

            lax.fori_loop(0, ct, tok_body, 0)
            w_cp[b] = pltpu.async_copy(
                out_v, out_hbm.at[pl.ds(wid * toks_per_w + it * ct, ct)], semw[b])
        for b in range(2):
            if w_cp[b] is not None:
                w_cp[b].wait()

    return k(yg, invA2d, invB2d)


def kernel(inputs, weight, gates, k, sorted_expert_idxs, sorted_scattered_idxs,
           padded_block_idxs, expert_offsets):
    n, d_in = inputs.shape
    e, d_out, _ = weight.shape
    kk = gates.shape[1]
    nk = n * kk

    ss2d = sorted_scattered_idxs.astype(jnp.int32).reshape(nk // 32, 32)
    xg, gg2d, inv = _sc_gather(inputs, gates.reshape(nk), ss2d, n, kk, d_in)
    yg = _grouped_mm(xg, weight, gg2d.reshape(nk), expert_offsets,
                     nk, d_in, d_out, e)
    inv_nk = inv.reshape(n, kk)
    invA2d = inv_nk[:, 0].reshape(n // 32, 32)
    invB2d = inv_nk[:, 1].reshape(n // 32, 32)
    return _sc_combine(yg, invA2d, invB2d, n, kk, d_out)
